# baseline (speedup 1.0000x reference)
import functools

import jax
import jax.numpy as jnp
from jax import lax
from jax.experimental import pallas as pl
from jax.experimental.pallas import tpu as pltpu
from jax.experimental.pallas import tpu_sc as plsc

N_NODES = 10000
CHANNELS = 128
NUM_LAYERS = 8
ALPHA = 0.1

NP = 10240
DUMMY = 10016
NC, NS = 2, 16
NW = NC * NS
ROWS_PER_TILE = NP // NS
K = 128
DEG_W = 128


def _deg_body(nchunks, cidx_hbm, zeros_hbm, ones_hbm, out_hbm,
              dshared, cidx_v, ones_v, sem):
    c = lax.axis_index("c")
    s = lax.axis_index("s")
    wid = s * NC + c
    base = s * ROWS_PER_TILE
    pltpu.sync_copy(zeros_hbm.at[pl.ds(base, ROWS_PER_TILE)],
                    dshared.at[pl.ds(base, ROWS_PER_TILE)])
    pltpu.sync_copy(ones_hbm, ones_v)
    pltpu.sync_copy(cidx_hbm.at[wid], cidx_v)
    plsc.subcore_barrier()

    cpp = nchunks // NPHASE

    for ph in range(NPHASE):
        def chunk(j, carry, ph=ph):
            pltpu.async_copy(ones_v, dshared.at[cidx_v.at[ph, j]], sem,
                             add=True)
            return carry

        lax.fori_loop(0, cpp, chunk, 0)

    for ph in range(NPHASE):
        def drain(j, carry, ph=ph):
            pltpu.make_async_copy(ones_v, dshared.at[cidx_v.at[ph, j]],
                                  sem).wait()
            return carry

        lax.fori_loop(0, cpp, drain, 0)
    plsc.subcore_barrier()
    pltpu.sync_copy(dshared.at[pl.ds(base, ROWS_PER_TILE)],
                    out_hbm.at[c, pl.ds(base, ROWS_PER_TILE)])


PIPE_D = 2
NPHASE = 2


def _scatter_body(nchunks, g_hbm, ridx_hbm, cidx_hbm, zeros_hbm, out_hbm,
                  sshared, ridx_v, cidx_v, rows_a, rows_b, gsem_a, gsem_b,
                  ssem_a, ssem_b):
    c = lax.axis_index("c")
    s = lax.axis_index("s")
    wid = s * NC + c
    base = s * ROWS_PER_TILE
    pltpu.sync_copy(zeros_hbm.at[pl.ds(base, ROWS_PER_TILE)],
                    sshared.at[pl.ds(base, ROWS_PER_TILE)])
    plsc.subcore_barrier()

    cpp = nchunks // NPHASE
    nouter = cpp // 2

    for ph in range(NPHASE):
        pltpu.sync_copy(ridx_hbm.at[wid, ph], ridx_v)
        pltpu.sync_copy(cidx_hbm.at[wid, ph], cidx_v)
        pltpu.async_copy(g_hbm.at[ridx_v.at[0]], rows_a, gsem_a)
        pltpu.async_copy(g_hbm.at[ridx_v.at[1]], rows_b, gsem_b)

        def outer(i, carry):
            j0 = i * 2
            pltpu.make_async_copy(g_hbm.at[ridx_v.at[j0]],
                                  rows_a, gsem_a).wait()
            sa = pltpu.async_copy(rows_a, sshared.at[cidx_v.at[j0]],
                                  ssem_a, add=True)
            pltpu.make_async_copy(g_hbm.at[ridx_v.at[j0 + 1]],
                                  rows_b, gsem_b).wait()
            sb = pltpu.async_copy(rows_b, sshared.at[cidx_v.at[j0 + 1]],
                                  ssem_b, add=True)
            sa.wait()
            pltpu.async_copy(g_hbm.at[ridx_v.at[j0 + 2]], rows_a, gsem_a)
            sb.wait()
            pltpu.async_copy(g_hbm.at[ridx_v.at[j0 + 3]], rows_b, gsem_b)
            return carry

        lax.fori_loop(0, nouter - 1, outer, 0)

        j0 = cpp - 2
        pltpu.make_async_copy(g_hbm.at[ridx_v.at[j0]],
                              rows_a, gsem_a).wait()
        sa = pltpu.async_copy(rows_a, sshared.at[cidx_v.at[j0]],
                              ssem_a, add=True)
        pltpu.make_async_copy(g_hbm.at[ridx_v.at[j0 + 1]],
                              rows_b, gsem_b).wait()
        sb = pltpu.async_copy(rows_b, sshared.at[cidx_v.at[j0 + 1]],
                              ssem_b, add=True)
        sa.wait()
        sb.wait()

    plsc.subcore_barrier()
    pltpu.sync_copy(sshared.at[pl.ds(base, ROWS_PER_TILE)],
                    out_hbm.at[c, pl.ds(base, ROWS_PER_TILE)])


def _init_tc_body(x_ref, wt_ref, b_ref, deg_ref, x0_ref, g_ref, dinvb_ref):
    h = jnp.dot(x_ref[...], wt_ref[...], preferred_element_type=jnp.float32)
    h = jnp.maximum(h + b_ref[...], 0.0)
    deg = deg_ref[0, :, 0:1] + deg_ref[1, :, 0:1]
    dinv = jnp.where(deg > 0, lax.rsqrt(deg), 0.0)
    dinvb = jnp.broadcast_to(dinv, (NP, CHANNELS))
    x0_ref[...] = h
    g_ref[...] = dinvb * h
    dinvb_ref[...] = dinvb


def _epi_tc_body(last, s_ref, x0_ref, dinvb_ref, h_ref, g_ref=None):
    dinvb = dinvb_ref[...]
    t = (1.0 - ALPHA) * (dinvb * (s_ref[0] + s_ref[1])) + ALPHA * x0_ref[...]
    if not last:
        t = jnp.maximum(t, 0.0)
        g_ref[...] = dinvb * t
    h_ref[...] = t


def kernel(x, edge_index, W_dense, b_dense):
    f32 = jnp.float32
    row = edge_index[0].astype(jnp.int32)
    col = edge_index[1].astype(jnp.int32)
    loop = jnp.arange(N_NODES, dtype=jnp.int32)
    rowf = jnp.concatenate([row, loop])
    colf = jnp.concatenate([col, loop])
    e_tot = rowf.shape[0]
    nchunks = -(-e_tot // (NW * K))
    nchunks = -(-nchunks // (PIPE_D * NPHASE)) * (PIPE_D * NPHASE)
    e_pad = NW * nchunks * K
    pad_n = e_pad - e_tot
    pad_iota = jnp.arange(pad_n, dtype=jnp.int32)
    pad_cols = N_NODES + pad_iota % (NP - N_NODES)
    rowp = jnp.concatenate([rowf, pad_iota % N_NODES])
    colp = jnp.concatenate([colf, pad_cols])
    ridx = jnp.transpose(rowp.reshape(NPHASE, nchunks // NPHASE, K, NW),
                         (3, 0, 1, 2))
    cidx = jnp.transpose(colp.reshape(NPHASE, nchunks // NPHASE, K, NW),
                         (3, 0, 1, 2))

    xp = jnp.zeros((NP, CHANNELS), f32).at[:N_NODES].set(x.astype(f32))
    zeros_c = jnp.zeros((NP, CHANNELS), f32)
    zeros_d = jnp.zeros((NP, DEG_W), f32)
    ones_d = jnp.ones((K, DEG_W), f32)

    mesh = plsc.VectorSubcoreMesh(
        core_axis_name="c", subcore_axis_name="s",
        num_cores=NC, num_subcores=NS)

    deg_kernel = pl.kernel(
        functools.partial(_deg_body, nchunks),
        out_type=jax.ShapeDtypeStruct((NC, NP, DEG_W), f32),
        mesh=mesh,
        scratch_types=[
            pltpu.VMEM_SHARED((NP, DEG_W), f32),
            pltpu.VMEM((NPHASE, nchunks // NPHASE, K), jnp.int32),
            pltpu.VMEM((K, DEG_W), f32),
            pltpu.SemaphoreType.DMA,
        ],
    )
    scatter_kernel = pl.kernel(
        functools.partial(_scatter_body, nchunks),
        out_type=jax.ShapeDtypeStruct((NC, NP, CHANNELS), f32),
        mesh=mesh,
        scratch_types=[
            pltpu.VMEM_SHARED((NP, CHANNELS), f32),
            pltpu.VMEM((nchunks // NPHASE, K), jnp.int32),
            pltpu.VMEM((nchunks // NPHASE, K), jnp.int32),
            pltpu.VMEM((K, CHANNELS), f32),
            pltpu.VMEM((K, CHANNELS), f32),
            pltpu.SemaphoreType.DMA,
            pltpu.SemaphoreType.DMA,
            pltpu.SemaphoreType.DMA,
            pltpu.SemaphoreType.DMA,
        ],
    )

    deg2 = deg_kernel(cidx, zeros_d, ones_d)

    init_tc = pl.pallas_call(
        _init_tc_body,
        out_shape=(
            jax.ShapeDtypeStruct((NP, CHANNELS), f32),
            jax.ShapeDtypeStruct((NP, CHANNELS), f32),
            jax.ShapeDtypeStruct((NP, CHANNELS), f32),
        ),
    )
    x0p, gp, dinvb = init_tc(xp, W_dense.T.astype(f32),
                             b_dense.astype(f32).reshape(1, CHANNELS), deg2)

    epi = pl.pallas_call(
        functools.partial(_epi_tc_body, False),
        out_shape=(
            jax.ShapeDtypeStruct((NP, CHANNELS), f32),
            jax.ShapeDtypeStruct((NP, CHANNELS), f32),
        ),
    )
    epi_last = pl.pallas_call(
        functools.partial(_epi_tc_body, True),
        out_shape=jax.ShapeDtypeStruct((NP, CHANNELS), f32),
    )

    h = x0p
    for layer in range(1, NUM_LAYERS + 1):
        s2 = scatter_kernel(gp, ridx, cidx, zeros_c)
        if layer < NUM_LAYERS:
            h, gp = epi(s2, x0p, dinvb)
        else:
            h = epi_last(s2, x0p, dinvb)
    return h[:N_NODES]


# speedup vs baseline: 1.1019x; 1.1019x over previous
import functools

import jax
import jax.numpy as jnp
from jax import lax
from jax.experimental import pallas as pl
from jax.experimental.pallas import tpu as pltpu
from jax.experimental.pallas import tpu_sc as plsc

N_NODES = 10000
CHANNELS = 128
NUM_LAYERS = 8
ALPHA = 0.1

NP = 10240
NC, NS = 2, 16
NW = NC * NS
ROWS_PER_TILE = NP // NS
K = 128
DEG_W = 128


def _deg_body(nchunks, cidx_hbm, zeros_hbm, ones_hbm, out_hbm,
              dshared, cidx_v, ones_v, sem):
    c = lax.axis_index("c")
    s = lax.axis_index("s")
    wid = s * NC + c
    base = s * ROWS_PER_TILE
    pltpu.sync_copy(zeros_hbm.at[pl.ds(base, ROWS_PER_TILE)],
                    dshared.at[pl.ds(base, ROWS_PER_TILE)])
    pltpu.sync_copy(ones_hbm, ones_v)
    pltpu.sync_copy(cidx_hbm.at[wid], cidx_v)
    plsc.subcore_barrier()

    cpp = nchunks // NPHASE

    for ph in range(NPHASE):
        def chunk(j, carry, ph=ph):
            pltpu.async_copy(ones_v, dshared.at[cidx_v.at[ph, j]], sem,
                             add=True)
            return carry

        lax.fori_loop(0, cpp, chunk, 0)

    for ph in range(NPHASE):
        def drain(j, carry, ph=ph):
            pltpu.make_async_copy(ones_v, dshared.at[cidx_v.at[ph, j]],
                                  sem).wait()
            return carry

        lax.fori_loop(0, cpp, drain, 0)
    plsc.subcore_barrier()
    pltpu.sync_copy(dshared.at[pl.ds(base, ROWS_PER_TILE)],
                    out_hbm.at[c, pl.ds(base, ROWS_PER_TILE)])


PIPE_D = 2
NPHASE = 2


def _scatter_body(nchunks, g_hbm, ridx_hbm, cidx_hbm, zeros_hbm, out_hbm,
                  sshared, ridx_v, cidx_v, rows_a, rows_b, gsem_a, gsem_b):
    c = lax.axis_index("c")
    s = lax.axis_index("s")
    wid = s * NC + c
    base = s * ROWS_PER_TILE
    pltpu.sync_copy(zeros_hbm.at[pl.ds(base, ROWS_PER_TILE)],
                    sshared.at[pl.ds(base, ROWS_PER_TILE)])
    plsc.subcore_barrier()

    cpp = nchunks // NPHASE
    nouter = cpp // 2

    for ph in range(NPHASE):
        pltpu.sync_copy(ridx_hbm.at[wid, ph], ridx_v)
        pltpu.sync_copy(cidx_hbm.at[wid, ph], cidx_v)
        pltpu.async_copy(g_hbm.at[ridx_v.at[0]], rows_a, gsem_a)

        def outer(i, carry):
            j0 = i * 2
            pltpu.make_async_copy(g_hbm.at[ridx_v.at[j0]],
                                  rows_a, gsem_a).wait()
            pltpu.async_copy(g_hbm.at[ridx_v.at[j0 + 1]], rows_b, gsem_b)
            pltpu.sync_copy(rows_a, sshared.at[cidx_v.at[j0]], add=True)
            pltpu.make_async_copy(g_hbm.at[ridx_v.at[j0 + 1]],
                                  rows_b, gsem_b).wait()
            pltpu.async_copy(g_hbm.at[ridx_v.at[j0 + 2]], rows_a, gsem_a)
            pltpu.sync_copy(rows_b, sshared.at[cidx_v.at[j0 + 1]],
                            add=True)
            return carry

        lax.fori_loop(0, nouter - 1, outer, 0)

        j0 = cpp - 2
        pltpu.make_async_copy(g_hbm.at[ridx_v.at[j0]],
                              rows_a, gsem_a).wait()
        pltpu.async_copy(g_hbm.at[ridx_v.at[j0 + 1]], rows_b, gsem_b)
        pltpu.sync_copy(rows_a, sshared.at[cidx_v.at[j0]], add=True)
        pltpu.make_async_copy(g_hbm.at[ridx_v.at[j0 + 1]],
                              rows_b, gsem_b).wait()
        pltpu.sync_copy(rows_b, sshared.at[cidx_v.at[j0 + 1]], add=True)

    plsc.subcore_barrier()
    pltpu.sync_copy(sshared.at[pl.ds(base, ROWS_PER_TILE)],
                    out_hbm.at[c, pl.ds(base, ROWS_PER_TILE)])


def _init_tc_body(x_ref, wt_ref, b_ref, deg_ref, x0_ref, g_ref, dinvb_ref):
    h = jnp.dot(x_ref[...], wt_ref[...], preferred_element_type=jnp.float32)
    h = jnp.maximum(h + b_ref[...], 0.0)
    deg = deg_ref[0, :, 0:1] + deg_ref[1, :, 0:1]
    dinv = jnp.where(deg > 0, lax.rsqrt(deg), 0.0)
    dinvb = jnp.broadcast_to(dinv, (NP, CHANNELS))
    x0_ref[...] = h
    g_ref[...] = dinvb * h
    dinvb_ref[...] = dinvb


def _epi_tc_body(last, s_ref, x0_ref, dinvb_ref, h_ref, g_ref=None):
    dinvb = dinvb_ref[...]
    t = (1.0 - ALPHA) * (dinvb * (s_ref[0] + s_ref[1])) + ALPHA * x0_ref[...]
    if not last:
        t = jnp.maximum(t, 0.0)
        g_ref[...] = dinvb * t
    h_ref[...] = t


def kernel(x, edge_index, W_dense, b_dense):
    f32 = jnp.float32
    row = edge_index[0].astype(jnp.int32)
    col = edge_index[1].astype(jnp.int32)
    loop = jnp.arange(N_NODES, dtype=jnp.int32)
    rowf = jnp.concatenate([row, loop])
    colf = jnp.concatenate([col, loop])
    e_tot = rowf.shape[0]
    nchunks = -(-e_tot // (NW * K))
    nchunks = -(-nchunks // (PIPE_D * NPHASE)) * (PIPE_D * NPHASE)
    e_pad = NW * nchunks * K
    pad_n = e_pad - e_tot
    pad_iota = jnp.arange(pad_n, dtype=jnp.int32)
    pad_cols = N_NODES + pad_iota % (NP - N_NODES)
    rowp = jnp.concatenate([rowf, pad_iota % N_NODES])
    colp = jnp.concatenate([colf, pad_cols])
    ridx = jnp.transpose(rowp.reshape(NPHASE, nchunks // NPHASE, K, NW),
                         (3, 0, 1, 2))
    cidx = jnp.transpose(colp.reshape(NPHASE, nchunks // NPHASE, K, NW),
                         (3, 0, 1, 2))

    xp = jnp.zeros((NP, CHANNELS), f32).at[:N_NODES].set(x.astype(f32))
    zeros_c = jnp.zeros((NP, CHANNELS), f32)
    zeros_d = jnp.zeros((NP, DEG_W), f32)
    ones_d = jnp.ones((K, DEG_W), f32)

    mesh = plsc.VectorSubcoreMesh(
        core_axis_name="c", subcore_axis_name="s",
        num_cores=NC, num_subcores=NS)

    deg_kernel = pl.kernel(
        functools.partial(_deg_body, nchunks),
        out_type=jax.ShapeDtypeStruct((NC, NP, DEG_W), f32),
        mesh=mesh,
        scratch_types=[
            pltpu.VMEM_SHARED((NP, DEG_W), f32),
            pltpu.VMEM((NPHASE, nchunks // NPHASE, K), jnp.int32),
            pltpu.VMEM((K, DEG_W), f32),
            pltpu.SemaphoreType.DMA,
        ],
    )
    scatter_kernel = pl.kernel(
        functools.partial(_scatter_body, nchunks),
        out_type=jax.ShapeDtypeStruct((NC, NP, CHANNELS), f32),
        mesh=mesh,
        scratch_types=[
            pltpu.VMEM_SHARED((NP, CHANNELS), f32),
            pltpu.VMEM((nchunks // NPHASE, K), jnp.int32),
            pltpu.VMEM((nchunks // NPHASE, K), jnp.int32),
            pltpu.VMEM((K, CHANNELS), f32),
            pltpu.VMEM((K, CHANNELS), f32),
            pltpu.SemaphoreType.DMA,
            pltpu.SemaphoreType.DMA,
        ],
    )

    deg2 = deg_kernel(cidx, zeros_d, ones_d)

    init_tc = pl.pallas_call(
        _init_tc_body,
        out_shape=(
            jax.ShapeDtypeStruct((NP, CHANNELS), f32),
            jax.ShapeDtypeStruct((NP, CHANNELS), f32),
            jax.ShapeDtypeStruct((NP, CHANNELS), f32),
        ),
    )
    x0p, gp, dinvb = init_tc(xp, W_dense.T.astype(f32),
                             b_dense.astype(f32).reshape(1, CHANNELS), deg2)

    epi = pl.pallas_call(
        functools.partial(_epi_tc_body, False),
        out_shape=(
            jax.ShapeDtypeStruct((NP, CHANNELS), f32),
            jax.ShapeDtypeStruct((NP, CHANNELS), f32),
        ),
    )
    epi_last = pl.pallas_call(
        functools.partial(_epi_tc_body, True),
        out_shape=jax.ShapeDtypeStruct((NP, CHANNELS), f32),
    )

    h = x0p
    for layer in range(1, NUM_LAYERS + 1):
        s2 = scatter_kernel(gp, ridx, cidx, zeros_c)
        if layer < NUM_LAYERS:
            h, gp = epi(s2, x0p, dinvb)
        else:
            h = epi_last(s2, x0p, dinvb)
    return h[:N_NODES]


# speedup vs baseline: 1.1101x; 1.0074x over previous
import functools

import jax
import jax.numpy as jnp
from jax import lax
from jax.experimental import pallas as pl
from jax.experimental.pallas import tpu as pltpu
from jax.experimental.pallas import tpu_sc as plsc

N_NODES = 10000
CHANNELS = 128
NUM_LAYERS = 8
ALPHA = 0.1

NP = 10240
NC, NS = 2, 16
NW = NC * NS
ROWS_PER_TILE = NP // NS
K = 128
DEG_W = 128


def _deg_body(nchunks, cidx_hbm, zeros_hbm, ones_hbm, out_hbm,
              dshared, cidx_v, ones_v, sem):
    c = lax.axis_index("c")
    s = lax.axis_index("s")
    wid = s * NC + c
    base = s * ROWS_PER_TILE
    pltpu.sync_copy(zeros_hbm.at[pl.ds(base, ROWS_PER_TILE)],
                    dshared.at[pl.ds(base, ROWS_PER_TILE)])
    pltpu.sync_copy(ones_hbm, ones_v)
    pltpu.sync_copy(cidx_hbm.at[wid], cidx_v)
    plsc.subcore_barrier()

    cpp = nchunks // NPHASE

    for ph in range(NPHASE):
        def chunk(j, carry, ph=ph):
            pltpu.async_copy(ones_v, dshared.at[cidx_v.at[ph, j]], sem,
                             add=True)
            return carry

        lax.fori_loop(0, cpp, chunk, 0)

    for ph in range(NPHASE):
        def drain(j, carry, ph=ph):
            pltpu.make_async_copy(ones_v, dshared.at[cidx_v.at[ph, j]],
                                  sem).wait()
            return carry

        lax.fori_loop(0, cpp, drain, 0)
    plsc.subcore_barrier()
    pltpu.sync_copy(dshared.at[pl.ds(base, ROWS_PER_TILE)],
                    out_hbm.at[c, pl.ds(base, ROWS_PER_TILE)])


PIPE_D = 2
NPHASE = 2


def _scatter_body(nchunks, g_hbm, ridx_hbm, cidx_hbm, zeros_hbm, out_hbm,
                  sshared, ridx_v, cidx_v, rows_a, rows_b, gsem_a, gsem_b):
    c = lax.axis_index("c")
    s = lax.axis_index("s")
    wid = s * NC + c
    base = s * ROWS_PER_TILE
    zd = pltpu.async_copy(zeros_hbm.at[pl.ds(base, ROWS_PER_TILE)],
                          sshared.at[pl.ds(base, ROWS_PER_TILE)], gsem_a)
    r0 = pltpu.async_copy(ridx_hbm.at[wid, 0], ridx_v, gsem_b)
    c0 = pltpu.async_copy(cidx_hbm.at[wid, 0], cidx_v, gsem_b)
    zd.wait()
    r0.wait()
    c0.wait()
    plsc.subcore_barrier()

    cpp = nchunks // NPHASE
    nouter = cpp // 2

    for ph in range(NPHASE):
        if ph > 0:
            pltpu.sync_copy(ridx_hbm.at[wid, ph], ridx_v)
            pltpu.sync_copy(cidx_hbm.at[wid, ph], cidx_v)
        pltpu.async_copy(g_hbm.at[ridx_v.at[0]], rows_a, gsem_a)

        def outer(i, carry):
            j0 = i * 2
            pltpu.make_async_copy(g_hbm.at[ridx_v.at[j0]],
                                  rows_a, gsem_a).wait()
            pltpu.async_copy(g_hbm.at[ridx_v.at[j0 + 1]], rows_b, gsem_b)
            pltpu.sync_copy(rows_a, sshared.at[cidx_v.at[j0]], add=True)
            pltpu.make_async_copy(g_hbm.at[ridx_v.at[j0 + 1]],
                                  rows_b, gsem_b).wait()
            pltpu.async_copy(g_hbm.at[ridx_v.at[j0 + 2]], rows_a, gsem_a)
            pltpu.sync_copy(rows_b, sshared.at[cidx_v.at[j0 + 1]],
                            add=True)
            return carry

        lax.fori_loop(0, nouter - 1, outer, 0)

        j0 = cpp - 2
        pltpu.make_async_copy(g_hbm.at[ridx_v.at[j0]],
                              rows_a, gsem_a).wait()
        pltpu.async_copy(g_hbm.at[ridx_v.at[j0 + 1]], rows_b, gsem_b)
        pltpu.sync_copy(rows_a, sshared.at[cidx_v.at[j0]], add=True)
        pltpu.make_async_copy(g_hbm.at[ridx_v.at[j0 + 1]],
                              rows_b, gsem_b).wait()
        pltpu.sync_copy(rows_b, sshared.at[cidx_v.at[j0 + 1]], add=True)

    plsc.subcore_barrier()
    pltpu.sync_copy(sshared.at[pl.ds(base, ROWS_PER_TILE)],
                    out_hbm.at[c, pl.ds(base, ROWS_PER_TILE)])


def _init_tc_body(x_ref, wt_ref, b_ref, deg_ref, x0_ref, g_ref, dinvb_ref):
    h = jnp.dot(x_ref[...], wt_ref[...], preferred_element_type=jnp.float32)
    h = jnp.maximum(h + b_ref[...], 0.0)
    deg = deg_ref[0, :, 0:1] + deg_ref[1, :, 0:1]
    dinv = jnp.where(deg > 0, lax.rsqrt(deg), 0.0)
    dinvb = jnp.broadcast_to(dinv, (NP, CHANNELS))
    x0_ref[...] = h
    g_ref[...] = dinvb * h
    dinvb_ref[...] = dinvb


def _epi_tc_body(last, s_ref, x0_ref, dinvb_ref, h_ref, g_ref=None):
    dinvb = dinvb_ref[...]
    t = (1.0 - ALPHA) * (dinvb * (s_ref[0] + s_ref[1])) + ALPHA * x0_ref[...]
    if not last:
        t = jnp.maximum(t, 0.0)
        g_ref[...] = dinvb * t
    h_ref[...] = t


def kernel(x, edge_index, W_dense, b_dense):
    f32 = jnp.float32
    row = edge_index[0].astype(jnp.int32)
    col = edge_index[1].astype(jnp.int32)
    loop = jnp.arange(N_NODES, dtype=jnp.int32)
    rowf = jnp.concatenate([row, loop])
    colf = jnp.concatenate([col, loop])
    e_tot = rowf.shape[0]
    nchunks = -(-e_tot // (NW * K))
    nchunks = -(-nchunks // (PIPE_D * NPHASE)) * (PIPE_D * NPHASE)
    e_pad = NW * nchunks * K
    pad_n = e_pad - e_tot
    pad_iota = jnp.arange(pad_n, dtype=jnp.int32)
    pad_cols = N_NODES + pad_iota % (NP - N_NODES)
    rowp = jnp.concatenate([rowf, pad_iota % N_NODES])
    colp = jnp.concatenate([colf, pad_cols])
    ridx = jnp.transpose(rowp.reshape(NPHASE, nchunks // NPHASE, K, NW),
                         (3, 0, 1, 2))
    cidx = jnp.transpose(colp.reshape(NPHASE, nchunks // NPHASE, K, NW),
                         (3, 0, 1, 2))

    xp = jnp.zeros((NP, CHANNELS), f32).at[:N_NODES].set(x.astype(f32))
    zeros_c = jnp.zeros((NP, CHANNELS), f32)
    zeros_d = jnp.zeros((NP, DEG_W), f32)
    ones_d = jnp.ones((K, DEG_W), f32)

    mesh = plsc.VectorSubcoreMesh(
        core_axis_name="c", subcore_axis_name="s",
        num_cores=NC, num_subcores=NS)

    deg_kernel = pl.kernel(
        functools.partial(_deg_body, nchunks),
        out_type=jax.ShapeDtypeStruct((NC, NP, DEG_W), f32),
        mesh=mesh,
        scratch_types=[
            pltpu.VMEM_SHARED((NP, DEG_W), f32),
            pltpu.VMEM((NPHASE, nchunks // NPHASE, K), jnp.int32),
            pltpu.VMEM((K, DEG_W), f32),
            pltpu.SemaphoreType.DMA,
        ],
    )
    scatter_kernel = pl.kernel(
        functools.partial(_scatter_body, nchunks),
        out_type=jax.ShapeDtypeStruct((NC, NP, CHANNELS), f32),
        mesh=mesh,
        scratch_types=[
            pltpu.VMEM_SHARED((NP, CHANNELS), f32),
            pltpu.VMEM((nchunks // NPHASE, K), jnp.int32),
            pltpu.VMEM((nchunks // NPHASE, K), jnp.int32),
            pltpu.VMEM((K, CHANNELS), f32),
            pltpu.VMEM((K, CHANNELS), f32),
            pltpu.SemaphoreType.DMA,
            pltpu.SemaphoreType.DMA,
        ],
    )

    deg2 = deg_kernel(cidx, zeros_d, ones_d)

    init_tc = pl.pallas_call(
        _init_tc_body,
        out_shape=(
            jax.ShapeDtypeStruct((NP, CHANNELS), f32),
            jax.ShapeDtypeStruct((NP, CHANNELS), f32),
            jax.ShapeDtypeStruct((NP, CHANNELS), f32),
        ),
    )
    x0p, gp, dinvb = init_tc(xp, W_dense.T.astype(f32),
                             b_dense.astype(f32).reshape(1, CHANNELS), deg2)

    epi = pl.pallas_call(
        functools.partial(_epi_tc_body, False),
        out_shape=(
            jax.ShapeDtypeStruct((NP, CHANNELS), f32),
            jax.ShapeDtypeStruct((NP, CHANNELS), f32),
        ),
    )
    epi_last = pl.pallas_call(
        functools.partial(_epi_tc_body, True),
        out_shape=jax.ShapeDtypeStruct((NP, CHANNELS), f32),
    )

    h = x0p
    for layer in range(1, NUM_LAYERS + 1):
        s2 = scatter_kernel(gp, ridx, cidx, zeros_c)
        if layer < NUM_LAYERS:
            h, gp = epi(s2, x0p, dinvb)
        else:
            h = epi_last(s2, x0p, dinvb)
    return h[:N_NODES]


# speedup vs baseline: 1.1139x; 1.0034x over previous
import functools

import jax
import jax.numpy as jnp
from jax import lax
from jax.experimental import pallas as pl
from jax.experimental.pallas import tpu as pltpu
from jax.experimental.pallas import tpu_sc as plsc

N_NODES = 10000
CHANNELS = 128
NUM_LAYERS = 8
ALPHA = 0.1

NP = 10240
NC, NS = 2, 16
NW = NC * NS
ROWS_PER_TILE = NP // NS
K = 128
DEG_W = 128


def _deg_body(nchunks, cidx_hbm, zeros_hbm, ones_hbm, out_hbm,
              dshared, cidx_v, ones_v, sem):
    c = lax.axis_index("c")
    s = lax.axis_index("s")
    wid = s * NC + c
    base = s * ROWS_PER_TILE
    zd = pltpu.async_copy(zeros_hbm.at[pl.ds(base, ROWS_PER_TILE)],
                          dshared.at[pl.ds(base, ROWS_PER_TILE)], sem)
    od = pltpu.async_copy(ones_hbm, ones_v, sem)
    cd = pltpu.async_copy(cidx_hbm.at[wid], cidx_v, sem)
    zd.wait()
    od.wait()
    cd.wait()
    plsc.subcore_barrier()

    cpp = nchunks // NPHASE

    for ph in range(NPHASE):
        def chunk(j, carry, ph=ph):
            pltpu.async_copy(ones_v, dshared.at[cidx_v.at[ph, j]], sem,
                             add=True)
            return carry

        lax.fori_loop(0, cpp, chunk, 0)

    for ph in range(NPHASE):
        def drain(j, carry, ph=ph):
            pltpu.make_async_copy(ones_v, dshared.at[cidx_v.at[ph, j]],
                                  sem).wait()
            return carry

        lax.fori_loop(0, cpp, drain, 0)
    plsc.subcore_barrier()
    pltpu.sync_copy(dshared.at[pl.ds(base, ROWS_PER_TILE)],
                    out_hbm.at[c, pl.ds(base, ROWS_PER_TILE)])


PIPE_D = 2
NPHASE = 2


def _scatter_body(nchunks, g_hbm, ridx_hbm, cidx_hbm, zeros_hbm, out_hbm,
                  sshared, ridx_v, cidx_v, rows_a, rows_b, gsem_a, gsem_b):
    c = lax.axis_index("c")
    s = lax.axis_index("s")
    wid = s * NC + c
    base = s * ROWS_PER_TILE
    zd = pltpu.async_copy(zeros_hbm.at[pl.ds(base, ROWS_PER_TILE)],
                          sshared.at[pl.ds(base, ROWS_PER_TILE)], gsem_a)
    r0 = pltpu.async_copy(ridx_hbm.at[wid, 0], ridx_v, gsem_b)
    c0 = pltpu.async_copy(cidx_hbm.at[wid, 0], cidx_v, gsem_b)
    zd.wait()
    r0.wait()
    c0.wait()
    plsc.subcore_barrier()

    cpp = nchunks // NPHASE
    nouter = cpp // 2

    for ph in range(NPHASE):
        if ph > 0:
            pltpu.sync_copy(ridx_hbm.at[wid, ph], ridx_v)
            pltpu.sync_copy(cidx_hbm.at[wid, ph], cidx_v)
        pltpu.async_copy(g_hbm.at[ridx_v.at[0]], rows_a, gsem_a)

        def outer(i, carry):
            j0 = i * 2
            pltpu.make_async_copy(g_hbm.at[ridx_v.at[j0]],
                                  rows_a, gsem_a).wait()
            pltpu.async_copy(g_hbm.at[ridx_v.at[j0 + 1]], rows_b, gsem_b)
            pltpu.sync_copy(rows_a, sshared.at[cidx_v.at[j0]], add=True)
            pltpu.make_async_copy(g_hbm.at[ridx_v.at[j0 + 1]],
                                  rows_b, gsem_b).wait()
            pltpu.async_copy(g_hbm.at[ridx_v.at[j0 + 2]], rows_a, gsem_a)
            pltpu.sync_copy(rows_b, sshared.at[cidx_v.at[j0 + 1]],
                            add=True)
            return carry

        lax.fori_loop(0, nouter - 1, outer, 0)

        j0 = cpp - 2
        pltpu.make_async_copy(g_hbm.at[ridx_v.at[j0]],
                              rows_a, gsem_a).wait()
        pltpu.async_copy(g_hbm.at[ridx_v.at[j0 + 1]], rows_b, gsem_b)
        pltpu.sync_copy(rows_a, sshared.at[cidx_v.at[j0]], add=True)
        pltpu.make_async_copy(g_hbm.at[ridx_v.at[j0 + 1]],
                              rows_b, gsem_b).wait()
        pltpu.sync_copy(rows_b, sshared.at[cidx_v.at[j0 + 1]], add=True)

    plsc.subcore_barrier()
    pltpu.sync_copy(sshared.at[pl.ds(base, ROWS_PER_TILE)],
                    out_hbm.at[c, pl.ds(base, ROWS_PER_TILE)])


def _init_tc_body(x_ref, wt_ref, b_ref, deg_ref, x0_ref, g_ref, dinvb_ref):
    h = jnp.dot(x_ref[...], wt_ref[...], preferred_element_type=jnp.float32)
    h = jnp.maximum(h + b_ref[...], 0.0)
    deg = deg_ref[0, :, 0:1] + deg_ref[1, :, 0:1]
    dinv = jnp.where(deg > 0, lax.rsqrt(deg), 0.0)
    dinvb = jnp.broadcast_to(dinv, (NP, CHANNELS))
    x0_ref[...] = h
    g_ref[...] = dinvb * h
    dinvb_ref[...] = dinvb


def _epi_tc_body(last, s_ref, x0_ref, dinvb_ref, h_ref, g_ref=None):
    dinvb = dinvb_ref[...]
    t = (1.0 - ALPHA) * (dinvb * (s_ref[0] + s_ref[1])) + ALPHA * x0_ref[...]
    if not last:
        t = jnp.maximum(t, 0.0)
        g_ref[...] = dinvb * t
    h_ref[...] = t


def kernel(x, edge_index, W_dense, b_dense):
    f32 = jnp.float32
    row = edge_index[0].astype(jnp.int32)
    col = edge_index[1].astype(jnp.int32)
    loop = jnp.arange(N_NODES, dtype=jnp.int32)
    rowf = jnp.concatenate([row, loop])
    colf = jnp.concatenate([col, loop])
    e_tot = rowf.shape[0]
    nchunks = -(-e_tot // (NW * K))
    nchunks = -(-nchunks // (PIPE_D * NPHASE)) * (PIPE_D * NPHASE)
    e_pad = NW * nchunks * K
    pad_n = e_pad - e_tot
    pad_iota = jnp.arange(pad_n, dtype=jnp.int32)
    pad_cols = N_NODES + pad_iota % (NP - N_NODES)
    rowp = jnp.concatenate([rowf, pad_iota % N_NODES])
    colp = jnp.concatenate([colf, pad_cols])
    ridx = jnp.transpose(rowp.reshape(NPHASE, nchunks // NPHASE, K, NW),
                         (3, 0, 1, 2))
    cidx = jnp.transpose(colp.reshape(NPHASE, nchunks // NPHASE, K, NW),
                         (3, 0, 1, 2))

    xp = jnp.zeros((NP, CHANNELS), f32).at[:N_NODES].set(x.astype(f32))
    zeros_c = jnp.zeros((NP, CHANNELS), f32)
    zeros_d = jnp.zeros((NP, DEG_W), f32)
    ones_d = jnp.ones((K, DEG_W), f32)

    mesh = plsc.VectorSubcoreMesh(
        core_axis_name="c", subcore_axis_name="s",
        num_cores=NC, num_subcores=NS)

    deg_kernel = pl.kernel(
        functools.partial(_deg_body, nchunks),
        out_type=jax.ShapeDtypeStruct((NC, NP, DEG_W), f32),
        mesh=mesh,
        scratch_types=[
            pltpu.VMEM_SHARED((NP, DEG_W), f32),
            pltpu.VMEM((NPHASE, nchunks // NPHASE, K), jnp.int32),
            pltpu.VMEM((K, DEG_W), f32),
            pltpu.SemaphoreType.DMA,
        ],
    )
    scatter_kernel = pl.kernel(
        functools.partial(_scatter_body, nchunks),
        out_type=jax.ShapeDtypeStruct((NC, NP, CHANNELS), f32),
        mesh=mesh,
        scratch_types=[
            pltpu.VMEM_SHARED((NP, CHANNELS), f32),
            pltpu.VMEM((nchunks // NPHASE, K), jnp.int32),
            pltpu.VMEM((nchunks // NPHASE, K), jnp.int32),
            pltpu.VMEM((K, CHANNELS), f32),
            pltpu.VMEM((K, CHANNELS), f32),
            pltpu.SemaphoreType.DMA,
            pltpu.SemaphoreType.DMA,
        ],
    )

    deg2 = deg_kernel(cidx, zeros_d, ones_d)

    init_tc = pl.pallas_call(
        _init_tc_body,
        out_shape=(
            jax.ShapeDtypeStruct((NP, CHANNELS), f32),
            jax.ShapeDtypeStruct((NP, CHANNELS), f32),
            jax.ShapeDtypeStruct((NP, CHANNELS), f32),
        ),
    )
    x0p, gp, dinvb = init_tc(xp, W_dense.T.astype(f32),
                             b_dense.astype(f32).reshape(1, CHANNELS), deg2)

    epi = pl.pallas_call(
        functools.partial(_epi_tc_body, False),
        out_shape=(
            jax.ShapeDtypeStruct((NP, CHANNELS), f32),
            jax.ShapeDtypeStruct((NP, CHANNELS), f32),
        ),
    )
    epi_last = pl.pallas_call(
        functools.partial(_epi_tc_body, True),
        out_shape=jax.ShapeDtypeStruct((NP, CHANNELS), f32),
    )

    h = x0p
    for layer in range(1, NUM_LAYERS + 1):
        s2 = scatter_kernel(gp, ridx, cidx, zeros_c)
        if layer < NUM_LAYERS:
            h, gp = epi(s2, x0p, dinvb)
        else:
            h = epi_last(s2, x0p, dinvb)
    return h[:N_NODES]
